# baseline (speedup 1.0000x reference)
import functools

import jax
import jax.numpy as jnp
from jax import lax
from jax.experimental import pallas as pl
from jax.experimental.pallas import tpu as pltpu
from jax.experimental.pallas import tpu_sc as plsc

NC = 2
NS = 16
LANES = 16
EB = 128
ROWBLK = 1000


def _split_blocks(nblk, nworkers, w):
    base, extra = nblk // nworkers, nblk % nworkers
    cnt = base + jnp.where(w < extra, 1, 0)
    lo = w * base + jnp.minimum(w, extra)
    return lo, cnt


def _make_degree(N, E):
    assert E % EB == 0
    nblk = E // EB
    mesh = plsc.VectorSubcoreMesh(core_axis_name="c", subcore_axis_name="s")

    @functools.partial(
        pl.kernel, mesh=mesh,
        out_type=jax.ShapeDtypeStruct((NC * NS, N), jnp.float32),
        compiler_params=pltpu.CompilerParams(needs_layout_passes=False),
        scratch_types=[
            pltpu.VMEM((N,), jnp.float32),
            pltpu.VMEM((EB,), jnp.int32),
        ],
    )
    def deg_kernel(dst_hbm, out_hbm, dloc, blk):
        c = lax.axis_index("c")
        s = lax.axis_index("s")
        wid = c * NS + s

        def zero_step(i, _):
            dloc[pl.ds(i * LANES, LANES)] = jnp.zeros((LANES,), jnp.float32)
            return 0
        lax.fori_loop(0, N // LANES, zero_step, 0)

        lo, cnt = _split_blocks(nblk, NC * NS, wid)
        ones = jnp.ones((LANES,), jnp.float32)

        def step(i, _):
            pltpu.sync_copy(dst_hbm.at[pl.ds((lo + i) * EB, EB)], blk)
            for j in range(EB // LANES):
                idx = blk[pl.ds(j * LANES, LANES)]
                plsc.addupdate_scatter(dloc, [idx], ones)
            return 0
        lax.fori_loop(0, cnt, step, 0)

        pltpu.sync_copy(dloc, out_hbm.at[wid])

    return deg_kernel


def _make_agg(N, E, M, Fc):
    assert E % EB == 0 and N % NS == 0
    nblk = E // EB
    rpt = N // NS
    G = 1
    SG = 10
    assert nblk % SG == 0 and SG % G == 0
    NG = SG // G
    nsg = nblk // SG
    nworkers = NC * NS
    out_shape = tuple(jax.ShapeDtypeStruct((NC, N, Fc), jnp.float32)
                      for _ in range(M))
    mesh = plsc.VectorSubcoreMesh(core_axis_name="c", subcore_axis_name="s")

    @functools.partial(
        pl.kernel, mesh=mesh,
        out_type=out_shape,
        compiler_params=pltpu.CompilerParams(use_tc_tiling_on_sc=False),
        scratch_types=[
            pltpu.VMEM_SHARED((N, Fc), jnp.float32),
            pltpu.VMEM((2, SG, EB), jnp.int32),
            pltpu.VMEM((2, SG, EB), jnp.int32),
            pltpu.VMEM((2, G, EB, Fc), jnp.float32),
            pltpu.SemaphoreType.DMA,
            pltpu.SemaphoreType.DMA,
            pltpu.SemaphoreType.DMA,
            pltpu.SemaphoreType.DMA,
        ],
    )
    def agg_kernel(*refs):
        z_refs = refs[:M]
        src_hbm, dst_hbm = refs[M], refs[M + 1]
        y_refs = refs[M + 2:2 * M + 2]
        acc, bidx, bdst, rows, sem_g, sem_s, sem_i0, sem_i1 = refs[2 * M + 2:]
        cid = lax.axis_index("c")
        sid = lax.axis_index("s")
        wid = cid * NS + sid
        slo, scnt = _split_blocks(nsg, nworkers, wid)
        isems = (sem_i0, sem_i1)

        def fire_idx(sgi, bank):
            b0 = sgi * SG
            pltpu.async_copy(src_hbm.at[pl.ds(b0, SG)], bidx.at[bank],
                             isems[bank])
            pltpu.async_copy(dst_hbm.at[pl.ds(b0, SG)], bdst.at[bank],
                             isems[bank])

        def wait_idx(bank):
            for _ in range(2):
                pltpu.make_async_copy(src_hbm.at[pl.ds(0, SG)],
                                      bidx.at[bank], isems[bank]).wait()

        for ci in range(M):
            zr = z_refs[ci]
            def zstep(i, _):
                for j in range(Fc // LANES):
                    rows[0, 0, i, pl.ds(j * LANES, LANES)] = (
                        jnp.zeros((LANES,), jnp.float32))
                return 0
            lax.fori_loop(0, EB, zstep, 0)
            nfull, tail = rpt // EB, rpt % EB
            for k in range(nfull):
                pltpu.sync_copy(
                    rows.at[0, 0], acc.at[pl.ds(sid * rpt + k * EB, EB)])
            if tail:
                pltpu.sync_copy(
                    rows.at[0, 0, pl.ds(0, tail)],
                    acc.at[pl.ds(sid * rpt + nfull * EB, tail)])

            plsc.subcore_barrier()

            def process(bank):
                pend = [[], []]
                for gg in range(NG):
                    rb = gg % 2
                    for d in pend[rb]:
                        d.wait()
                    pend[rb] = []
                    gd = [pltpu.async_copy(
                        zr.at[bidx.at[bank, gg * G + j]],
                        rows.at[rb, j], sem_g) for j in range(G)]
                    for j in range(G):
                        gd[j].wait()
                        pend[rb].append(pltpu.async_copy(
                            rows.at[rb, j],
                            acc.at[bdst.at[bank, gg * G + j]],
                            sem_s, add=True))
                for rb in range(2):
                    for d in pend[rb]:
                        d.wait()

            @pl.when(scnt > 0)
            def _():
                fire_idx(slo, 0)

            def pair(p, _):
                sg0 = slo + 2 * p

                @pl.when(2 * p + 1 < scnt)
                def _():
                    fire_idx(sg0 + 1, 1)
                wait_idx(0)
                process(0)

                @pl.when(2 * p + 1 < scnt)
                def _():
                    @pl.when(2 * p + 2 < scnt)
                    def _():
                        fire_idx(sg0 + 2, 0)
                    wait_idx(1)
                    process(1)
                return 0
            lax.fori_loop(0, (scnt + 1) // 2, pair, 0)

            plsc.subcore_barrier()

            pltpu.sync_copy(acc.at[pl.ds(sid * rpt, rpt)],
                            y_refs[ci].at[cid, pl.ds(sid * rpt, rpt)])

    return agg_kernel


def _tc_call(body, grid, in_specs, out_specs, out_shape):
    return pl.pallas_call(
        body, grid=grid, in_specs=in_specs, out_specs=out_specs,
        out_shape=out_shape)


def _dinv_from(dp):
    return lax.rsqrt(jnp.sum(dp, axis=1, keepdims=True) + 1.0)


def _tc1(dpT, x, W1, N, R):
    H = W1.shape[1]

    def body(dp_ref, x_ref, w_ref, o_ref):
        dv = _dinv_from(dp_ref[...])
        o_ref[...] = jnp.dot(x_ref[...] * dv, w_ref[...],
                             preferred_element_type=jnp.float32)

    return _tc_call(
        body, (N // R,),
        [pl.BlockSpec((R, NC * NS), lambda i: (i, 0)),
         pl.BlockSpec((R, x.shape[1]), lambda i: (i, 0)),
         pl.BlockSpec(W1.shape, lambda i: (0, 0))],
        pl.BlockSpec((R, H), lambda i: (i, 0)),
        jax.ShapeDtypeStruct((N, H), jnp.float32),
    )(dpT, x, W1)


def _tc2(p1, g1, dpT, b1, N, R):
    H = g1.shape[1]

    def body(p_ref, g_ref, dp_ref, b_ref, o_ref):
        dv = _dinv_from(dp_ref[...])
        a = p_ref[0] + p_ref[1] + g_ref[...]
        o_ref[...] = dv * jnp.maximum(dv * a + b_ref[...], 0.0)

    return _tc_call(
        body, (N // R,),
        [pl.BlockSpec((2, R, H), lambda i: (0, i, 0)),
         pl.BlockSpec((R, H), lambda i: (i, 0)),
         pl.BlockSpec((R, NC * NS), lambda i: (i, 0)),
         pl.BlockSpec((1, H), lambda i: (0, 0))],
        pl.BlockSpec((R, H), lambda i: (i, 0)),
        jax.ShapeDtypeStruct((N, H), jnp.float32),
    )(p1, g1, dpT, b1.reshape(1, -1))


def _tc3(p2, z2, dpT, W2, b2, W3, N, R):
    H = z2.shape[1]
    H3 = W3.shape[1]
    Fc = H3 // 4

    def body(p_ref, z_ref, dp_ref, w2_ref, b2_ref, w3_ref, *o_refs):
        dv = _dinv_from(dp_ref[...])
        a = (p_ref[0] + p_ref[1] + z_ref[...]) * dv
        t = jnp.maximum(jnp.dot(a, w2_ref[...],
                                preferred_element_type=jnp.float32)
                        + b2_ref[...], 0.0)
        g = jnp.dot(t * dv, w3_ref[...], preferred_element_type=jnp.float32)
        for c in range(4):
            o_refs[c][...] = g[:, c * Fc:(c + 1) * Fc]

    return _tc_call(
        body, (N // R,),
        [pl.BlockSpec((2, R, H), lambda i: (0, i, 0)),
         pl.BlockSpec((R, H), lambda i: (i, 0)),
         pl.BlockSpec((R, NC * NS), lambda i: (i, 0)),
         pl.BlockSpec(W2.shape, lambda i: (0, 0)),
         pl.BlockSpec((1, W2.shape[1]), lambda i: (0, 0)),
         pl.BlockSpec(W3.shape, lambda i: (0, 0))],
        [pl.BlockSpec((R, Fc), lambda i: (i, 0)) for _ in range(4)],
        [jax.ShapeDtypeStruct((N, Fc), jnp.float32) for _ in range(4)],
    )(p2, z2, dpT, W2, b2.reshape(1, -1), W3)


def _tc4(p3s, g3s, dpT, b3, W4, N, R):
    Fc = g3s[0].shape[1]
    OUT = W4.shape[1]
    b3c = b3.reshape(4, 1, Fc)

    def body(*refs):
        p_refs = refs[0:4]
        g_refs = refs[4:8]
        dp_ref, b_ref, w4_ref, o_ref = refs[8:]
        dv = _dinv_from(dp_ref[...])
        h = jnp.concatenate(
            [jnp.maximum(dv * (p_refs[c][0] + p_refs[c][1] + g_refs[c][...])
                         + b_ref[c], 0.0) for c in range(4)],
            axis=1)
        o_ref[...] = jnp.dot(h * dv, w4_ref[...],
                             preferred_element_type=jnp.float32)

    return _tc_call(
        body, (N // R,),
        [pl.BlockSpec((2, R, Fc), lambda i: (0, i, 0)) for _ in range(4)]
        + [pl.BlockSpec((R, Fc), lambda i: (i, 0)) for _ in range(4)]
        + [pl.BlockSpec((R, NC * NS), lambda i: (i, 0)),
           pl.BlockSpec((4, 1, Fc), lambda i: (0, 0, 0)),
           pl.BlockSpec(W4.shape, lambda i: (0, 0))],
        pl.BlockSpec((R, OUT), lambda i: (i, 0)),
        jax.ShapeDtypeStruct((N, OUT), jnp.float32),
    )(*p3s, *g3s, dpT, b3c, W4)


def _tc5(parts, g4, dpT, b4, N, R):
    OUT = g4.shape[1]

    def body(p_ref, g_ref, dp_ref, b_ref, o_ref):
        dv = _dinv_from(dp_ref[...])
        o = dv * (p_ref[0] + p_ref[1] + g_ref[...]) + b_ref[...]
        m = jnp.max(o, axis=1, keepdims=True)
        e = o - m
        o_ref[...] = e - jnp.log(jnp.sum(jnp.exp(e), axis=1, keepdims=True))

    return _tc_call(
        body, (N // R,),
        [pl.BlockSpec((2, R, OUT), lambda i: (0, i, 0)),
         pl.BlockSpec((R, OUT), lambda i: (i, 0)),
         pl.BlockSpec((R, NC * NS), lambda i: (i, 0)),
         pl.BlockSpec((1, OUT), lambda i: (0, 0))],
        pl.BlockSpec((R, OUT), lambda i: (i, 0)),
        jax.ShapeDtypeStruct((N, OUT), jnp.float32),
    )(parts, g4, dpT, b4.reshape(1, -1))


def kernel(x, edge_index, W1, b1, W2, b2, W3, b3, W4, b4):
    N, DIN = x.shape
    E = edge_index.shape[1]
    R = ROWBLK
    src = edge_index[0]
    dst = edge_index[1]
    nblk = E // EB
    src2 = src.reshape(nblk, EB)
    dst2 = dst.reshape(nblk, EB)

    deg_parts = _make_degree(N, E)(dst)
    dpT = jnp.transpose(deg_parts)

    agg128 = _make_agg(N, E, 1, W1.shape[1])
    agg512 = _make_agg(N, E, 4, W3.shape[1] // 4)
    agg16 = _make_agg(N, E, 1, W4.shape[1])

    g1 = _tc1(dpT, x, W1, N, R)
    (p1,) = agg128(g1, src2, dst2)
    z2 = _tc2(p1, g1, dpT, b1, N, R)
    (p2,) = agg128(z2, src2, dst2)
    g3s = _tc3(p2, z2, dpT, W2, b2, W3, N, R)
    p3s = agg512(*g3s, src2, dst2)
    g4 = _tc4(p3s, g3s, dpT, b3, W4, N, R)
    (parts,) = agg16(g4, src2, dst2)
    return _tc5(parts, g4, dpT, b4, N, R)
